# baseline (speedup 1.0000x reference)
import functools

import jax
import jax.numpy as jnp
from jax import lax
from jax.experimental import pallas as pl
from jax.experimental.pallas import tpu as pltpu
from jax.experimental.pallas import tpu_sc as plsc

_EPS = 1e-07
_VOCAB = 30522
_NTOK = 1024
_NC, _NS, _L = 2, 16, 16
_NW = _NC * _NS
_BW = _NTOK // _NW


def _sc_gather(y_pred, yt):
    mesh = plsc.VectorSubcoreMesh(core_axis_name="c", subcore_axis_name="s")

    @functools.partial(
        pl.kernel,
        mesh=mesh,
        out_type=jax.ShapeDtypeStruct((_NTOK,), jnp.float32),
        scratch_types=[
            pltpu.VMEM((_BW,), jnp.int32),
            pltpu.VMEM((_BW, 8, 128), jnp.float32),
            pltpu.VMEM((_BW,), jnp.float32),
            pltpu.SemaphoreType.DMA,
        ],
        compiler_params=pltpu.CompilerParams(
            use_tc_tiling_on_sc=True, needs_layout_passes=False
        ),
    )
    def body(yp_hbm, yt_hbm, out_hbm, yt_v, tiles_v, p_v, sem):
        wid = lax.axis_index("s") * _NC + lax.axis_index("c")
        base = wid * _BW
        pltpu.sync_copy(yt_hbm.at[pl.ds(base, _BW)], yt_v)
        lanes = lax.iota(jnp.int32, _L)
        copies = []
        for j in range(_BW):
            b = (base + j) // 128
            r8 = ((base + j) % 128) // 8 * 8
            cvec = (yt_v[pl.ds((j // _L) * _L, _L)] >> 7) << 7
            c = jnp.sum(jnp.where(lanes == (j % _L), cvec, 0))
            copies.append(
                pltpu.make_async_copy(
                    yp_hbm.at[b].at[
                        pl.ds(pl.multiple_of(r8, 8), 8),
                        pl.ds(pl.multiple_of(c, 128), 128),
                    ],
                    tiles_v.at[j],
                    sem,
                )
            )
            copies[-1].start()
        for cp in copies:
            cp.wait()
        for v in range(_BW // _L):
            tok = lax.iota(jnp.int32, _L) + v * _L
            sub = (base + tok) % 8
            lane = yt_v[pl.ds(v * _L, _L)] & 127
            p_v[pl.ds(v * _L, _L)] = plsc.load_gather(tiles_v, [tok, sub, lane])
        pltpu.sync_copy(p_v, out_hbm.at[pl.ds(base, _BW)])

    return body(y_pred, yt)


def _loss_body(p_ref, yt_ref, um_ref, out_ref):
    yt = yt_ref[...]
    um = um_ref[...]
    p = p_ref[...]
    cnt = jnp.sum((yt == um).astype(jnp.float32), axis=1, keepdims=True)
    alpha = jnp.where(
        (yt >= 2) & (cnt > 0.0),
        lax.rsqrt(jnp.maximum(cnt, 1e-20)),
        0.0,
    )
    keep = (yt != 0).astype(jnp.float32)
    a = alpha * keep
    pc = jnp.clip(p, _EPS, 1.0 - _EPS)
    om = 1.0 - pc
    num = jnp.sum(a * om * om * (-jnp.log(pc)))
    den = jnp.sum(a)
    out_ref[...] = (num / den).reshape(1, 1)


def kernel(y_pred, y_true):
    yt = y_true[:, :, 0].reshape(-1)
    um = y_true[:, :, 1].reshape(-1)
    p = _sc_gather(y_pred, yt)
    out = pl.pallas_call(
        _loss_body,
        out_shape=jax.ShapeDtypeStruct((1, 1), jnp.float32),
    )(
        p.reshape(_NTOK, 1),
        yt.reshape(_NTOK, 1),
        um.reshape(1, _NTOK),
    )
    return out[0, 0]


# speedup vs baseline: 5.4231x; 5.4231x over previous
import functools

import jax
import jax.numpy as jnp
from jax import lax
from jax.experimental import pallas as pl
from jax.experimental.pallas import tpu as pltpu
from jax.experimental.pallas import tpu_sc as plsc

_EPS = 1e-07
_VOCAB = 30522
_NTOK = 1024
_NC, _NS, _L = 2, 16, 16
_NW = _NC * _NS
_BW = _NTOK // _NW


def _sc_gather(yp_rows, yt):
    mesh = plsc.VectorSubcoreMesh(core_axis_name="c", subcore_axis_name="s")

    @functools.partial(
        pl.kernel,
        mesh=mesh,
        out_type=jax.ShapeDtypeStruct((_NTOK,), jnp.float32),
        scratch_types=[
            pltpu.VMEM((_BW,), jnp.int32),
            pltpu.VMEM((_BW,), jnp.int32),
            pltpu.VMEM((_BW, 128), jnp.float32),
            pltpu.VMEM((_BW,), jnp.float32),
            pltpu.SemaphoreType.DMA,
        ],
        compiler_params=pltpu.CompilerParams(needs_layout_passes=False),
    )
    def body(yp_hbm, yt_hbm, out_hbm, yt_v, idx_v, rows_v, p_v, sem):
        wid = lax.axis_index("s") * _NC + lax.axis_index("c")
        base = wid * _BW
        b = wid // 4
        pltpu.sync_copy(yt_hbm.at[pl.ds(base, _BW)], yt_v)
        for v in range(_BW // _L):
            sl = pl.ds(v * _L, _L)
            idx_v[sl] = yt_v[sl] * 8 + b
        pltpu.async_copy(yp_hbm.at[idx_v], rows_v, sem).wait()
        for v in range(_BW // _L):
            tok = lax.iota(jnp.int32, _L) + v * _L
            lane = (base + tok) % 128
            p_v[pl.ds(v * _L, _L)] = plsc.load_gather(rows_v, [tok, lane])
        pltpu.sync_copy(p_v, out_hbm.at[pl.ds(base, _BW)])

    return body(yp_rows, yt)


def _loss_body(p_ref, yt_ref, um_ref, out_ref):
    yt = yt_ref[...]
    um = um_ref[...]
    p = p_ref[...]
    cnt = jnp.sum((yt == um).astype(jnp.float32), axis=1, keepdims=True)
    alpha = jnp.where(
        (yt >= 2) & (cnt > 0.0),
        lax.rsqrt(jnp.maximum(cnt, 1e-20)),
        0.0,
    )
    keep = (yt != 0).astype(jnp.float32)
    a = alpha * keep
    pc = jnp.clip(p, _EPS, 1.0 - _EPS)
    om = 1.0 - pc
    num = jnp.sum(a * om * om * (-jnp.log(pc)))
    den = jnp.sum(a)
    out_ref[...] = (num / den).reshape(1, 1)


def kernel(y_pred, y_true):
    yt = y_true[:, :, 0].reshape(-1)
    um = y_true[:, :, 1].reshape(-1)
    yp_rows = y_pred.transpose(2, 0, 1).reshape(_VOCAB * 8, 128)
    p = _sc_gather(yp_rows, yt)
    out = pl.pallas_call(
        _loss_body,
        out_shape=jax.ShapeDtypeStruct((1, 1), jnp.float32),
    )(
        p.reshape(_NTOK, 1),
        yt.reshape(_NTOK, 1),
        um.reshape(1, _NTOK),
    )
    return out[0, 0]
